# baseline (speedup 1.0000x reference)
import jax
import jax.numpy as jnp
from jax.experimental import pallas as pl
from jax.experimental.pallas import tpu as pltpu

N_EXPERTS = 8
D_MODEL = 768
D_FF = 768
N_TOKENS = 2048
THRESHOLD = 0.1
TOKEN_TILE = 256


def _gate_body(x_ref, wg_ref, bg_ref, coeff_ref):
    logits = jnp.dot(x_ref[:], wg_ref[:], preferred_element_type=jnp.float32)
    logits = logits + bg_ref[:]
    mx = jnp.max(logits, axis=1, keepdims=True)
    ex = jnp.exp(logits - mx)
    w = ex / jnp.sum(ex, axis=1, keepdims=True)
    act = w > THRESHOLD
    act_any = jnp.any(act, axis=0, keepdims=True)
    any_act = jnp.any(act)
    coeff_ref[:] = jnp.where(any_act, w * act_any.astype(jnp.float32),
                             1.0 / N_EXPERTS)


def _moe_body(x_ref, w1_ref, b1_ref, w2_ref, b2_ref, cf_ref, out_ref):
    x = x_ref[:].astype(jnp.bfloat16)
    cf = cf_ref[:]
    acc = None
    for e in range(N_EXPERTS):
        h = jnp.dot(x, w1_ref[e], preferred_element_type=jnp.float32)
        h = jnp.maximum(h + b1_ref[e:e + 1, :], 0.0).astype(jnp.bfloat16)
        o = jnp.dot(h, w2_ref[e], preferred_element_type=jnp.float32)
        o = o + b2_ref[e:e + 1, :]
        term = cf[:, e:e + 1] * o
        acc = term if acc is None else acc + term
    out_ref[:] = acc


@jax.jit
def kernel(t, x, W1, b1, W2, b2, Wg, bg):
    del t
    n_tiles = N_TOKENS // TOKEN_TILE
    wg_x = Wg[:D_MODEL]
    bg2 = bg.reshape(1, N_EXPERTS)
    W1 = W1.astype(jnp.bfloat16)
    W2 = W2.astype(jnp.bfloat16)

    coeff = pl.pallas_call(
        _gate_body,
        out_shape=jax.ShapeDtypeStruct((N_TOKENS, N_EXPERTS), jnp.float32),
    )(x, wg_x, bg2)

    out = pl.pallas_call(
        _moe_body,
        grid=(n_tiles,),
        in_specs=[
            pl.BlockSpec((TOKEN_TILE, D_MODEL), lambda i: (i, 0)),
            pl.BlockSpec((N_EXPERTS, D_MODEL, D_FF), lambda i: (0, 0, 0)),
            pl.BlockSpec((N_EXPERTS, D_FF), lambda i: (0, 0)),
            pl.BlockSpec((N_EXPERTS, D_FF, D_MODEL), lambda i: (0, 0, 0)),
            pl.BlockSpec((N_EXPERTS, D_MODEL), lambda i: (0, 0)),
            pl.BlockSpec((TOKEN_TILE, N_EXPERTS), lambda i: (i, 0)),
        ],
        out_specs=pl.BlockSpec((TOKEN_TILE, D_MODEL), lambda i: (i, 0)),
        out_shape=jax.ShapeDtypeStruct((N_TOKENS, D_MODEL), jnp.float32),
        compiler_params=pltpu.CompilerParams(
            dimension_semantics=("parallel",)),
    )(x, W1, b1, W2, b2, coeff)
    return out


# speedup vs baseline: 1.2874x; 1.2874x over previous
import jax
import jax.numpy as jnp
from jax.experimental import pallas as pl
from jax.experimental.pallas import tpu as pltpu

N_EXPERTS = 8
D_MODEL = 768
D_FF = 768
N_TOKENS = 2048
THRESHOLD = 0.1
TOKEN_TILE = 256


def _moe_body(x_ref, w1_ref, b1_ref, w2_ref, b2_ref, wg_ref, bg_ref,
              out_ref, coeff_ref):
    t = pl.program_id(0)

    @pl.when(t == 0)
    def _gate():
        xx = x_ref[:]
        logits = jnp.dot(xx, wg_ref[:], preferred_element_type=jnp.float32)
        logits = logits + bg_ref[:]
        mx = jnp.max(logits, axis=1, keepdims=True)
        ex = jnp.exp(logits - mx)
        w = ex / jnp.sum(ex, axis=1, keepdims=True)
        act = w > THRESHOLD
        act_any = jnp.any(act, axis=0, keepdims=True)
        any_act = jnp.any(act)
        coeff_ref[:] = jnp.where(any_act, w * act_any.astype(jnp.float32),
                                 1.0 / N_EXPERTS)

    rows = pl.ds(t * TOKEN_TILE, TOKEN_TILE)
    x = x_ref[rows, :]
    cf = coeff_ref[rows, :]
    acc = None
    for e in range(N_EXPERTS):
        h = jnp.dot(x, w1_ref[e], preferred_element_type=jnp.float32)
        h = jnp.maximum(h + b1_ref[e:e + 1, :], 0.0)
        o = jnp.dot(h, w2_ref[e], preferred_element_type=jnp.float32)
        o = o + b2_ref[e:e + 1, :]
        term = cf[:, e:e + 1] * o
        acc = term if acc is None else acc + term
    out_ref[:] = acc


@jax.jit
def kernel(t, x, W1, b1, W2, b2, Wg, bg):
    del t
    n_tiles = N_TOKENS // TOKEN_TILE
    wg_x = Wg[:D_MODEL]
    bg2 = bg.reshape(1, N_EXPERTS)

    out = pl.pallas_call(
        _moe_body,
        grid=(n_tiles,),
        in_specs=[
            pl.BlockSpec((N_TOKENS, D_MODEL), lambda i: (0, 0)),
            pl.BlockSpec((N_EXPERTS, D_MODEL, D_FF), lambda i: (0, 0, 0)),
            pl.BlockSpec((N_EXPERTS, D_FF), lambda i: (0, 0)),
            pl.BlockSpec((N_EXPERTS, D_FF, D_MODEL), lambda i: (0, 0, 0)),
            pl.BlockSpec((N_EXPERTS, D_MODEL), lambda i: (0, 0)),
            pl.BlockSpec((D_MODEL, N_EXPERTS), lambda i: (0, 0)),
            pl.BlockSpec((1, N_EXPERTS), lambda i: (0, 0)),
        ],
        out_specs=pl.BlockSpec((TOKEN_TILE, D_MODEL), lambda i: (i, 0)),
        out_shape=jax.ShapeDtypeStruct((N_TOKENS, D_MODEL), jnp.float32),
        scratch_shapes=[
            pltpu.VMEM((N_TOKENS, N_EXPERTS), jnp.float32),
        ],
    )(x, W1, b1, W2, b2, wg_x, bg2)
    return out
